# baseline (speedup 1.0000x reference)
import jax
import jax.numpy as jnp
from jax.experimental import pallas as pl
from jax.experimental.pallas import tpu as pltpu

_N = 20000
_CONF = 0.35
_IOU = 0.5
_MAXDET = 1000
_NR, _NC = 256, 128
_BR, _BC = 8, 128
_BSZ = _BR * _BC
_NPAD = _NR * _NC


def _nms_body(x1_ref, y1_ref, x2_ref, y2_ref, sc_ref,
              ocx_ref, ocy_ref, ow_ref, oh_ref, osc_ref, ov_ref,
              k_ref, sx1_ref, sy1_ref, sx2_ref, sy2_ref):
    f = (jax.lax.broadcasted_iota(jnp.int32, (_NR, _NC), 0) * _NC
         + jax.lax.broadcasted_iota(jnp.int32, (_NR, _NC), 1))

    sc = sc_ref[...]
    key = jnp.where(sc > _CONF, sc, -1.0)
    idx = f
    x1 = x1_ref[...]
    y1 = y1_ref[...]
    x2 = x2_ref[...]
    y2 = y2_ref[...]

    def _exchange(s, kk, j, fetch):
        key, idx, x1, y1, x2, y2 = s
        lob = (f & j) == 0
        pk = fetch(key, lob)
        pi = fetch(idx, lob)
        pless = (pk > key) | ((pk == key) & (pi < idx))
        dirdesc = (f & kk) != 0
        take = jnp.logical_xor(jnp.logical_xor(pless, lob),
                               jnp.logical_not(dirdesc))
        return (jnp.where(take, pk, key),
                jnp.where(take, pi, idx),
                jnp.where(take, fetch(x1, lob), x1),
                jnp.where(take, fetch(y1, lob), y1),
                jnp.where(take, fetch(x2, lob), x2),
                jnp.where(take, fetch(y2, lob), y2))

    def _mk_branch(dr):
        def br(kk, *s):
            def fetch(x, lob):
                r = x.reshape(_NR // (2 * dr), 2, dr, _NC)
                return jnp.concatenate([r[:, 1:2], r[:, 0:1]],
                                       axis=1).reshape(_NR, _NC)

            return _exchange(s, kk, dr * _NC, fetch)

        return br

    _branches = [_mk_branch(1 << t) for t in range(8)]

    def _sub_body(c):
        kk, j, di = c[0], c[1], c[2]
        s = jax.lax.switch(di, _branches, kk, *c[3:])
        return (kk, jax.lax.shift_right_logical(j, 1), di - 1, *s)

    def _lane_body(c):
        kk, j = c[0], c[1]

        def fetch(x, lob):
            return jnp.where(lob, pltpu.roll(x, _NC - j, axis=1),
                             pltpu.roll(x, j, axis=1))

        return (kk, jax.lax.shift_right_logical(j, 1), c[2],
                *_exchange(c[3:], kk, j, fetch))

    def _level_body(lv, s):
        kk = jax.lax.shift_left(jnp.int32(1), lv)
        j0 = jax.lax.shift_right_logical(kk, 1)
        c = jax.lax.while_loop(lambda t: t[1] >= _NC, _sub_body,
                               (kk, j0, lv - 8) + s)
        c = jax.lax.while_loop(lambda t: t[1] >= 1, _lane_body, c)
        return c[3:]

    res = jax.lax.fori_loop(1, 16, _level_body,
                            (key, idx, x1, y1, x2, y2))
    key, _, x1, y1, x2, y2 = res

    k_ref[...] = key
    sx1_ref[...] = x1
    sy1_ref[...] = y1
    sx2_ref[...] = x2
    sy2_ref[...] = y2

    _U = 8
    g = (jax.lax.broadcasted_iota(jnp.int32, (_BR, _BC), 0) * _BC
         + jax.lax.broadcasted_iota(jnp.int32, (_BR, _BC), 1))
    zf = jnp.zeros((_BR, _BC), jnp.float32)
    zi = jnp.zeros((_BR, _BC), jnp.int32)

    sel0 = f == 0
    s0 = jnp.sum(jnp.where(sel0, key, 0.0))

    def cond(c):
        return (c[1] > 0.0) & (c[2] < _MAXDET)

    def body(c):
        (p, s_next, ks, kv, c_sc, c_x1, c_y1, c_x2, c_y2,
         kx1, ky1, kx2, ky2, ka, ocx, ocy, ow, oh, osc, ov) = c
        base = p & (_NC - 1)

        def ext(ch, o):
            ssh = (_BR - jax.lax.shift_right_logical(o, 7)) & (_BR - 1)
            lsh = (_NC - (o & (_NC - 1))) & (_NC - 1)
            r = pltpu.roll(ch, ssh, axis=0)
            r = pltpu.roll(r, lsh, axis=1)
            return jnp.broadcast_to(r[0:1, 0:1], (_BR, _BC))

        for u in range(_U):
            o = base + u
            sv = ext(c_sc, o)
            bx1 = ext(c_x1, o)
            by1 = ext(c_y1, o)
            bx2 = ext(c_x2, o)
            by2 = ext(c_y2, o)
            a1 = (bx2 - bx1) * (by2 - by1)
            ix1 = jnp.maximum(bx1, kx1)
            iy1 = jnp.maximum(by1, ky1)
            ix2 = jnp.minimum(bx2, kx2)
            iy2 = jnp.minimum(by2, ky2)
            inter = (jnp.maximum(ix2 - ix1, 0.0)
                     * jnp.maximum(iy2 - iy1, 0.0))
            iou = inter / (a1 + ka - inter + 1e-9)
            gt = jnp.where(iou > _IOU, 1.0, 0.0)
            t = gt
            for sh in (64, 32, 16, 8, 4, 2, 1):
                t = jnp.maximum(t, pltpu.roll(t, sh, axis=1))
            for sh in (4, 2, 1):
                t = jnp.maximum(t, pltpu.roll(t, sh, axis=0))
            okv = (t < 0.5) & (sv > 0.0) & (kv < _MAXDET)
            slot = (g == kv) & okv
            kx1 = jnp.where(slot, bx1, kx1)
            ky1 = jnp.where(slot, by1, ky1)
            kx2 = jnp.where(slot, bx2, kx2)
            ky2 = jnp.where(slot, by2, ky2)
            ka = jnp.where(slot, a1, ka)
            w = bx2 - bx1
            h = by2 - by1
            ocx = jnp.where(slot, bx1 + w / 2.0, ocx)
            ocy = jnp.where(slot, by1 + h / 2.0, ocy)
            ow = jnp.where(slot, w, ow)
            oh = jnp.where(slot, h, oh)
            osc = jnp.where(slot, sv, osc)
            ov = jnp.where(slot, 1.0, ov)
            kv = kv + jnp.where(okv, 1, 0)

        pn = p + _U
        sn = jnp.max(ext(c_sc, base + _U)[0:1, 0:1])
        ksn = jnp.max(kv[0:1, 0:1])
        r0 = jax.lax.shift_right_logical(pn, 7)
        n_sc = k_ref[pl.ds(r0, _BR), :]
        n_x1 = sx1_ref[pl.ds(r0, _BR), :]
        n_y1 = sy1_ref[pl.ds(r0, _BR), :]
        n_x2 = sx2_ref[pl.ds(r0, _BR), :]
        n_y2 = sy2_ref[pl.ds(r0, _BR), :]
        return (pn, sn, ksn, kv, n_sc, n_x1, n_y1, n_x2, n_y2,
                kx1, ky1, kx2, ky2, ka, ocx, ocy, ow, oh, osc, ov)

    init = (jnp.int32(0), s0, jnp.int32(0), zi,
            key[0:_BR], x1[0:_BR], y1[0:_BR], x2[0:_BR], y2[0:_BR],
            zf, zf, zf, zf, zf, zf, zf, zf, zf, zf, zf)
    res = jax.lax.while_loop(cond, body, init)
    ocx_ref[...] = res[14]
    ocy_ref[...] = res[15]
    ow_ref[...] = res[16]
    oh_ref[...] = res[17]
    osc_ref[...] = res[18]
    ov_ref[...] = res[19]


def kernel(boxes, scores):
    pad = _NPAD - _N
    shp = (_NR, _NC)
    x1 = jnp.pad(boxes[:, 0], (0, pad)).reshape(shp)
    y1 = jnp.pad(boxes[:, 1], (0, pad)).reshape(shp)
    x2 = jnp.pad(boxes[:, 2], (0, pad)).reshape(shp)
    y2 = jnp.pad(boxes[:, 3], (0, pad)).reshape(shp)
    sc = jnp.pad(scores, (0, pad)).reshape(shp)
    outs = pl.pallas_call(
        _nms_body,
        out_shape=[jax.ShapeDtypeStruct((_BR, _BC), jnp.float32)] * 6,
        scratch_shapes=[pltpu.VMEM(shp, jnp.float32)] * 5,
    )(x1, y1, x2, y2, sc)
    cols = [o.reshape(-1)[:_MAXDET] for o in outs]
    return jnp.stack(cols, axis=-1)


# speedup vs baseline: 1.0110x; 1.0110x over previous
import jax
import jax.numpy as jnp
from jax.experimental import pallas as pl
from jax.experimental.pallas import tpu as pltpu

_N = 20000
_CONF = 0.35
_IOU = 0.5
_MAXDET = 1000
_NR, _NC = 256, 128
_BR, _BC = 8, 128
_BSZ = _BR * _BC
_NPAD = _NR * _NC


def _nms_body(x1_ref, y1_ref, x2_ref, y2_ref, sc_ref,
              ocx_ref, ocy_ref, ow_ref, oh_ref, osc_ref, ov_ref,
              k_ref, sx1_ref, sy1_ref, sx2_ref, sy2_ref):
    f = (jax.lax.broadcasted_iota(jnp.int32, (_NR, _NC), 0) * _NC
         + jax.lax.broadcasted_iota(jnp.int32, (_NR, _NC), 1))

    sc = sc_ref[...]
    key = jnp.where(sc > _CONF, sc, -1.0)
    idx = f
    x1 = x1_ref[...]
    y1 = y1_ref[...]
    x2 = x2_ref[...]
    y2 = y2_ref[...]

    def _exchange(s, kk, j, fetch):
        key, idx, x1, y1, x2, y2 = s
        lob = (f & j) == 0
        pk = fetch(key, lob)
        pi = fetch(idx, lob)
        pless = (pk > key) | ((pk == key) & (pi < idx))
        dirdesc = (f & kk) != 0
        take = jnp.logical_xor(jnp.logical_xor(pless, lob),
                               jnp.logical_not(dirdesc))
        return (jnp.where(take, pk, key),
                jnp.where(take, pi, idx),
                jnp.where(take, fetch(x1, lob), x1),
                jnp.where(take, fetch(y1, lob), y1),
                jnp.where(take, fetch(x2, lob), x2),
                jnp.where(take, fetch(y2, lob), y2))

    def _mk_branch(dr):
        def br(kk, *s):
            def fetch(x, lob):
                r = x.reshape(_NR // (2 * dr), 2, dr, _NC)
                return jnp.concatenate([r[:, 1:2], r[:, 0:1]],
                                       axis=1).reshape(_NR, _NC)

            return _exchange(s, kk, dr * _NC, fetch)

        return br

    _branches = [_mk_branch(1 << t) for t in range(8)]

    def _sub_body(c):
        kk, j, di = c[0], c[1], c[2]
        s = jax.lax.switch(di, _branches, kk, *c[3:])
        return (kk, jax.lax.shift_right_logical(j, 1), di - 1, *s)

    def _lane_body(c):
        kk, j = c[0], c[1]

        def fetch(x, lob):
            return jnp.where(lob, pltpu.roll(x, _NC - j, axis=1),
                             pltpu.roll(x, j, axis=1))

        return (kk, jax.lax.shift_right_logical(j, 1), c[2],
                *_exchange(c[3:], kk, j, fetch))

    def _level_body(lv, s):
        kk = jax.lax.shift_left(jnp.int32(1), lv)
        j0 = jax.lax.shift_right_logical(kk, 1)
        c = jax.lax.while_loop(lambda t: t[1] >= _NC, _sub_body,
                               (kk, j0, lv - 8) + s)
        c = jax.lax.while_loop(lambda t: t[1] >= 1, _lane_body, c)
        return c[3:]

    res = jax.lax.fori_loop(1, 16, _level_body,
                            (key, idx, x1, y1, x2, y2))
    key, _, x1, y1, x2, y2 = res

    k_ref[...] = key
    sx1_ref[...] = x1
    sy1_ref[...] = y1
    sx2_ref[...] = x2
    sy2_ref[...] = y2

    _U = 8
    g = (jax.lax.broadcasted_iota(jnp.int32, (_BR, _BC), 0) * _BC
         + jax.lax.broadcasted_iota(jnp.int32, (_BR, _BC), 1))
    zf = jnp.zeros((_BR, _BC), jnp.float32)
    zi = jnp.zeros((_BR, _BC), jnp.int32)

    sel0 = f == 0
    s0 = jnp.sum(jnp.where(sel0, key, 0.0))

    def cond(c):
        return (c[1] > 0.0) & (c[2] < _MAXDET)

    def body(c):
        (p, s_next, ks, kv, c_sc, c_x1, c_y1, c_x2, c_y2,
         kx1, ky1, kx2, ky2, ka, ocx, ocy, ow, oh, osc, ov) = c

        def ext(w, u):
            return jnp.broadcast_to(w[0:1, u:u + 1], (_BR, _BC))

        for u in range(_U):
            sv = ext(c_sc, u)
            bx1 = ext(c_x1, u)
            by1 = ext(c_y1, u)
            bx2 = ext(c_x2, u)
            by2 = ext(c_y2, u)
            a1 = (bx2 - bx1) * (by2 - by1)
            ix1 = jnp.maximum(bx1, kx1)
            iy1 = jnp.maximum(by1, ky1)
            ix2 = jnp.minimum(bx2, kx2)
            iy2 = jnp.minimum(by2, ky2)
            inter = (jnp.maximum(ix2 - ix1, 0.0)
                     * jnp.maximum(iy2 - iy1, 0.0))
            iou = inter / (a1 + ka - inter + 1e-9)
            gt = jnp.where(iou > _IOU, 1.0, 0.0)
            t = gt
            for sh in (64, 32, 16, 8, 4, 2, 1):
                t = jnp.maximum(t, pltpu.roll(t, sh, axis=1))
            for sh in (4, 2, 1):
                t = jnp.maximum(t, pltpu.roll(t, sh, axis=0))
            okv = (t < 0.5) & (sv > 0.0) & (kv < _MAXDET)
            slot = (g == kv) & okv
            kx1 = jnp.where(slot, bx1, kx1)
            ky1 = jnp.where(slot, by1, ky1)
            kx2 = jnp.where(slot, bx2, kx2)
            ky2 = jnp.where(slot, by2, ky2)
            ka = jnp.where(slot, a1, ka)
            w = bx2 - bx1
            h = by2 - by1
            ocx = jnp.where(slot, bx1 + w / 2.0, ocx)
            ocy = jnp.where(slot, by1 + h / 2.0, ocy)
            ow = jnp.where(slot, w, ow)
            oh = jnp.where(slot, h, oh)
            osc = jnp.where(slot, sv, osc)
            ov = jnp.where(slot, 1.0, ov)
            kv = kv + jnp.where(okv, 1, 0)

        pn = p + _U
        ksn = jnp.max(kv[0:1, 0:1])
        r0 = jax.lax.shift_right_logical(pn, 7)
        lsh = (_NC - (pn & (_NC - 1))) & (_NC - 1)
        n_sc = pltpu.roll(k_ref[pl.ds(r0, 1), :], lsh, axis=1)
        n_x1 = pltpu.roll(sx1_ref[pl.ds(r0, 1), :], lsh, axis=1)
        n_y1 = pltpu.roll(sy1_ref[pl.ds(r0, 1), :], lsh, axis=1)
        n_x2 = pltpu.roll(sx2_ref[pl.ds(r0, 1), :], lsh, axis=1)
        n_y2 = pltpu.roll(sy2_ref[pl.ds(r0, 1), :], lsh, axis=1)
        sn = jnp.max(n_sc[0:1, 0:1])
        return (pn, sn, ksn, kv, n_sc, n_x1, n_y1, n_x2, n_y2,
                kx1, ky1, kx2, ky2, ka, ocx, ocy, ow, oh, osc, ov)

    init = (jnp.int32(0), s0, jnp.int32(0), zi,
            key[0:1], x1[0:1], y1[0:1], x2[0:1], y2[0:1],
            zf, zf, zf, zf, zf, zf, zf, zf, zf, zf, zf)
    res = jax.lax.while_loop(cond, body, init)
    ocx_ref[...] = res[14]
    ocy_ref[...] = res[15]
    ow_ref[...] = res[16]
    oh_ref[...] = res[17]
    osc_ref[...] = res[18]
    ov_ref[...] = res[19]


def kernel(boxes, scores):
    pad = _NPAD - _N
    shp = (_NR, _NC)
    x1 = jnp.pad(boxes[:, 0], (0, pad)).reshape(shp)
    y1 = jnp.pad(boxes[:, 1], (0, pad)).reshape(shp)
    x2 = jnp.pad(boxes[:, 2], (0, pad)).reshape(shp)
    y2 = jnp.pad(boxes[:, 3], (0, pad)).reshape(shp)
    sc = jnp.pad(scores, (0, pad)).reshape(shp)
    outs = pl.pallas_call(
        _nms_body,
        out_shape=[jax.ShapeDtypeStruct((_BR, _BC), jnp.float32)] * 6,
        scratch_shapes=[pltpu.VMEM(shp, jnp.float32)] * 5,
    )(x1, y1, x2, y2, sc)
    cols = [o.reshape(-1)[:_MAXDET] for o in outs]
    return jnp.stack(cols, axis=-1)


# speedup vs baseline: 1.0207x; 1.0096x over previous
import jax
import jax.numpy as jnp
from jax.experimental import pallas as pl
from jax.experimental.pallas import tpu as pltpu

_N = 20000
_CONF = 0.35
_IOU = 0.5
_MAXDET = 1000
_NR, _NC = 256, 128
_BR, _BC = 8, 128
_BSZ = _BR * _BC
_NPAD = _NR * _NC


def _nms_body(x1_ref, y1_ref, x2_ref, y2_ref, sc_ref,
              ocx_ref, ocy_ref, ow_ref, oh_ref, osc_ref, ov_ref,
              k_ref, sx1_ref, sy1_ref, sx2_ref, sy2_ref):
    f = (jax.lax.broadcasted_iota(jnp.int32, (_NR, _NC), 0) * _NC
         + jax.lax.broadcasted_iota(jnp.int32, (_NR, _NC), 1))

    sc = sc_ref[...]
    key = jnp.where(sc > _CONF, sc, -1.0)
    idx = f
    x1 = x1_ref[...]
    y1 = y1_ref[...]
    x2 = x2_ref[...]
    y2 = y2_ref[...]

    def _exchange(s, kk, j, fetch):
        key, idx, x1, y1, x2, y2 = s
        lob = (f & j) == 0
        pk = fetch(key, lob)
        pi = fetch(idx, lob)
        pless = (pk > key) | ((pk == key) & (pi < idx))
        dirdesc = (f & kk) != 0
        take = jnp.logical_xor(jnp.logical_xor(pless, lob),
                               jnp.logical_not(dirdesc))
        return (jnp.where(take, pk, key),
                jnp.where(take, pi, idx),
                jnp.where(take, fetch(x1, lob), x1),
                jnp.where(take, fetch(y1, lob), y1),
                jnp.where(take, fetch(x2, lob), x2),
                jnp.where(take, fetch(y2, lob), y2))

    def _mk_branch(dr):
        def br(kk, *s):
            def fetch(x, lob):
                r = x.reshape(_NR // (2 * dr), 2, dr, _NC)
                return jnp.concatenate([r[:, 1:2], r[:, 0:1]],
                                       axis=1).reshape(_NR, _NC)

            return _exchange(s, kk, dr * _NC, fetch)

        return br

    _branches = [_mk_branch(1 << t) for t in range(8)]

    def _sub_body(c):
        kk, j, di = c[0], c[1], c[2]
        s = jax.lax.switch(di, _branches, kk, *c[3:])
        return (kk, jax.lax.shift_right_logical(j, 1), di - 1, *s)

    def _lane_body(c):
        kk, j = c[0], c[1]

        def fetch(x, lob):
            return jnp.where(lob, pltpu.roll(x, _NC - j, axis=1),
                             pltpu.roll(x, j, axis=1))

        return (kk, jax.lax.shift_right_logical(j, 1), c[2],
                *_exchange(c[3:], kk, j, fetch))

    def _level_body(lv, s):
        kk = jax.lax.shift_left(jnp.int32(1), lv)
        j0 = jax.lax.shift_right_logical(kk, 1)
        c = jax.lax.while_loop(lambda t: t[1] >= _NC, _sub_body,
                               (kk, j0, lv - 8) + s)
        c = jax.lax.while_loop(lambda t: t[1] >= 1, _lane_body, c)
        return c[3:]

    res = jax.lax.fori_loop(1, 16, _level_body,
                            (key, idx, x1, y1, x2, y2))
    key, _, x1, y1, x2, y2 = res

    k_ref[...] = key
    sx1_ref[...] = x1
    sy1_ref[...] = y1
    sx2_ref[...] = x2
    sy2_ref[...] = y2

    _U = 8
    g = (jax.lax.broadcasted_iota(jnp.int32, (_BR, _BC), 0) * _BC
         + jax.lax.broadcasted_iota(jnp.int32, (_BR, _BC), 1))
    zf = jnp.zeros((_BR, _BC), jnp.float32)
    zi = jnp.zeros((_BR, _BC), jnp.int32)

    sel0 = f == 0
    s0 = jnp.sum(jnp.where(sel0, key, 0.0))

    def cond(c):
        return (c[1] > 0.0) & (c[2] < _MAXDET)

    def body(c):
        (p, s_next, ks, kv, c_sc, c_x1, c_y1, c_x2, c_y2,
         kx1, ky1, kx2, ky2, ka, ocx, ocy, ow, oh, osc, ov) = c

        def ext(w, u):
            return jnp.broadcast_to(w[0:1, u:u + 1], (_BR, _BC))

        for u in range(_U):
            sv = ext(c_sc, u)
            bx1 = ext(c_x1, u)
            by1 = ext(c_y1, u)
            bx2 = ext(c_x2, u)
            by2 = ext(c_y2, u)
            a1 = (bx2 - bx1) * (by2 - by1)
            ix1 = jnp.maximum(bx1, kx1)
            iy1 = jnp.maximum(by1, ky1)
            ix2 = jnp.minimum(bx2, kx2)
            iy2 = jnp.minimum(by2, ky2)
            inter = (jnp.maximum(ix2 - ix1, 0.0)
                     * jnp.maximum(iy2 - iy1, 0.0))
            den = a1 + ka - inter + 1e-9
            gt = jnp.where(inter - den * 0.5 > den * (2.0 ** -25),
                           1.0, 0.0)
            t = gt
            for sh in (64, 32, 16, 8, 4, 2, 1):
                t = jnp.maximum(t, pltpu.roll(t, sh, axis=1))
            for sh in (4, 2, 1):
                t = jnp.maximum(t, pltpu.roll(t, sh, axis=0))
            okv = (t < 0.5) & (sv > 0.0) & (kv < _MAXDET)
            slot = (g == kv) & okv
            kx1 = jnp.where(slot, bx1, kx1)
            ky1 = jnp.where(slot, by1, ky1)
            kx2 = jnp.where(slot, bx2, kx2)
            ky2 = jnp.where(slot, by2, ky2)
            ka = jnp.where(slot, a1, ka)
            w = bx2 - bx1
            h = by2 - by1
            ocx = jnp.where(slot, bx1 + w / 2.0, ocx)
            ocy = jnp.where(slot, by1 + h / 2.0, ocy)
            ow = jnp.where(slot, w, ow)
            oh = jnp.where(slot, h, oh)
            osc = jnp.where(slot, sv, osc)
            ov = jnp.where(slot, 1.0, ov)
            kv = kv + jnp.where(okv, 1, 0)

        pn = p + _U
        ksn = jnp.max(kv[0:1, 0:1])
        r0 = jax.lax.shift_right_logical(pn, 7)
        lsh = (_NC - (pn & (_NC - 1))) & (_NC - 1)
        n_sc = pltpu.roll(k_ref[pl.ds(r0, 1), :], lsh, axis=1)
        n_x1 = pltpu.roll(sx1_ref[pl.ds(r0, 1), :], lsh, axis=1)
        n_y1 = pltpu.roll(sy1_ref[pl.ds(r0, 1), :], lsh, axis=1)
        n_x2 = pltpu.roll(sx2_ref[pl.ds(r0, 1), :], lsh, axis=1)
        n_y2 = pltpu.roll(sy2_ref[pl.ds(r0, 1), :], lsh, axis=1)
        sn = jnp.max(n_sc[0:1, 0:1])
        return (pn, sn, ksn, kv, n_sc, n_x1, n_y1, n_x2, n_y2,
                kx1, ky1, kx2, ky2, ka, ocx, ocy, ow, oh, osc, ov)

    init = (jnp.int32(0), s0, jnp.int32(0), zi,
            key[0:1], x1[0:1], y1[0:1], x2[0:1], y2[0:1],
            zf, zf, zf, zf, zf, zf, zf, zf, zf, zf, zf)
    res = jax.lax.while_loop(cond, body, init)
    ocx_ref[...] = res[14]
    ocy_ref[...] = res[15]
    ow_ref[...] = res[16]
    oh_ref[...] = res[17]
    osc_ref[...] = res[18]
    ov_ref[...] = res[19]


def kernel(boxes, scores):
    pad = _NPAD - _N
    shp = (_NR, _NC)
    x1 = jnp.pad(boxes[:, 0], (0, pad)).reshape(shp)
    y1 = jnp.pad(boxes[:, 1], (0, pad)).reshape(shp)
    x2 = jnp.pad(boxes[:, 2], (0, pad)).reshape(shp)
    y2 = jnp.pad(boxes[:, 3], (0, pad)).reshape(shp)
    sc = jnp.pad(scores, (0, pad)).reshape(shp)
    outs = pl.pallas_call(
        _nms_body,
        out_shape=[jax.ShapeDtypeStruct((_BR, _BC), jnp.float32)] * 6,
        scratch_shapes=[pltpu.VMEM(shp, jnp.float32)] * 5,
    )(x1, y1, x2, y2, sc)
    cols = [o.reshape(-1)[:_MAXDET] for o in outs]
    return jnp.stack(cols, axis=-1)


# speedup vs baseline: 1.7682x; 1.7323x over previous
import jax
import jax.numpy as jnp
from jax.experimental import pallas as pl
from jax.experimental.pallas import tpu as pltpu

_N = 20000
_CONF = 0.35
_IOU = 0.5
_MAXDET = 1000
_NR, _NC = 256, 128
_BR, _BC = 8, 128
_BSZ = _BR * _BC
_NPAD = _NR * _NC


def _nms_body(x1_ref, y1_ref, x2_ref, y2_ref, sc_ref,
              ocx_ref, ocy_ref, ow_ref, oh_ref, osc_ref, ov_ref,
              k_ref, sx1_ref, sy1_ref, sx2_ref, sy2_ref):
    f = (jax.lax.broadcasted_iota(jnp.int32, (_NR, _NC), 0) * _NC
         + jax.lax.broadcasted_iota(jnp.int32, (_NR, _NC), 1))

    sc = sc_ref[...]
    key = jnp.where(sc > _CONF, sc, -1.0)
    idx = f
    x1 = x1_ref[...]
    y1 = y1_ref[...]
    x2 = x2_ref[...]
    y2 = y2_ref[...]

    def _exchange(s, kk, j, fetch):
        key, idx, x1, y1, x2, y2 = s
        lob = (f & j) == 0
        pk = fetch(key, lob)
        pi = fetch(idx, lob)
        pless = (pk > key) | ((pk == key) & (pi < idx))
        dirdesc = (f & kk) != 0
        take = jnp.logical_xor(jnp.logical_xor(pless, lob),
                               jnp.logical_not(dirdesc))
        return (jnp.where(take, pk, key),
                jnp.where(take, pi, idx),
                jnp.where(take, fetch(x1, lob), x1),
                jnp.where(take, fetch(y1, lob), y1),
                jnp.where(take, fetch(x2, lob), x2),
                jnp.where(take, fetch(y2, lob), y2))

    def _mk_branch(dr):
        def br(kk, *s):
            def fetch(x, lob):
                r = x.reshape(_NR // (2 * dr), 2, dr, _NC)
                return jnp.concatenate([r[:, 1:2], r[:, 0:1]],
                                       axis=1).reshape(_NR, _NC)

            return _exchange(s, kk, dr * _NC, fetch)

        return br

    _branches = [_mk_branch(1 << t) for t in range(8)]

    def _sub_body(c):
        kk, j, di = c[0], c[1], c[2]
        s = jax.lax.switch(di, _branches, kk, *c[3:])
        return (kk, jax.lax.shift_right_logical(j, 1), di - 1, *s)

    def _lane_body(c):
        kk, j = c[0], c[1]

        def fetch(x, lob):
            return jnp.where(lob, pltpu.roll(x, _NC - j, axis=1),
                             pltpu.roll(x, j, axis=1))

        return (kk, jax.lax.shift_right_logical(j, 1), c[2],
                *_exchange(c[3:], kk, j, fetch))

    def _level_body(lv, s):
        kk = jax.lax.shift_left(jnp.int32(1), lv)
        j0 = jax.lax.shift_right_logical(kk, 1)
        c = jax.lax.while_loop(lambda t: t[1] >= _NC, _sub_body,
                               (kk, j0, lv - 8) + s)
        c = jax.lax.while_loop(lambda t: t[1] >= 1, _lane_body, c)
        return c[3:]

    res = jax.lax.fori_loop(1, 16, _level_body,
                            (key, idx, x1, y1, x2, y2))
    key, _, x1, y1, x2, y2 = res

    k_ref[...] = key
    sx1_ref[...] = x1
    sy1_ref[...] = y1
    sx2_ref[...] = x2
    sy2_ref[...] = y2

    _U = 8
    g = (jax.lax.broadcasted_iota(jnp.int32, (_BR, _BC), 0) * _BC
         + jax.lax.broadcasted_iota(jnp.int32, (_BR, _BC), 1))
    zf = jnp.zeros((_BR, _BC), jnp.float32)
    zi = jnp.zeros((_BR, _BC), jnp.int32)
    ones_l = jnp.ones((_BC, _BC), jnp.float32)
    ones_s = jnp.ones((_BR, _BR), jnp.float32)

    sel0 = f == 0
    s0 = jnp.sum(jnp.where(sel0, key, 0.0))

    def cond(c):
        return (c[1] > 0.0) & (c[2] < _MAXDET)

    def body(c):
        (p, s_next, ks, kv, c_sc, c_x1, c_y1, c_x2, c_y2,
         kx1, ky1, kx2, ky2, ka, ocx, ocy, ow, oh, osc, ov) = c

        def ext(w, u):
            return jnp.broadcast_to(w[0:1, u:u + 1], (_BR, _BC))

        for u in range(_U):
            sv = ext(c_sc, u)
            bx1 = ext(c_x1, u)
            by1 = ext(c_y1, u)
            bx2 = ext(c_x2, u)
            by2 = ext(c_y2, u)
            a1 = (bx2 - bx1) * (by2 - by1)
            ix1 = jnp.maximum(bx1, kx1)
            iy1 = jnp.maximum(by1, ky1)
            ix2 = jnp.minimum(bx2, kx2)
            iy2 = jnp.minimum(by2, ky2)
            inter = (jnp.maximum(ix2 - ix1, 0.0)
                     * jnp.maximum(iy2 - iy1, 0.0))
            den = a1 + ka - inter + 1e-9
            gt = jnp.where(inter - den * 0.5 > den * (2.0 ** -25),
                           1.0, 0.0)
            rs = jax.lax.dot_general(
                gt, ones_l, (((1,), (0,)), ((), ())),
                preferred_element_type=jnp.float32)
            t = jax.lax.dot_general(
                ones_s, rs, (((1,), (0,)), ((), ())),
                preferred_element_type=jnp.float32)
            okv = (t < 0.5) & (sv > 0.0) & (kv < _MAXDET)
            slot = (g == kv) & okv
            kx1 = jnp.where(slot, bx1, kx1)
            ky1 = jnp.where(slot, by1, ky1)
            kx2 = jnp.where(slot, bx2, kx2)
            ky2 = jnp.where(slot, by2, ky2)
            ka = jnp.where(slot, a1, ka)
            w = bx2 - bx1
            h = by2 - by1
            ocx = jnp.where(slot, bx1 + w / 2.0, ocx)
            ocy = jnp.where(slot, by1 + h / 2.0, ocy)
            ow = jnp.where(slot, w, ow)
            oh = jnp.where(slot, h, oh)
            osc = jnp.where(slot, sv, osc)
            ov = jnp.where(slot, 1.0, ov)
            kv = kv + jnp.where(okv, 1, 0)

        pn = p + _U
        ksn = jnp.max(kv[0:1, 0:1])
        r0 = jax.lax.shift_right_logical(pn, 7)
        lsh = (_NC - (pn & (_NC - 1))) & (_NC - 1)
        n_sc = pltpu.roll(k_ref[pl.ds(r0, 1), :], lsh, axis=1)
        n_x1 = pltpu.roll(sx1_ref[pl.ds(r0, 1), :], lsh, axis=1)
        n_y1 = pltpu.roll(sy1_ref[pl.ds(r0, 1), :], lsh, axis=1)
        n_x2 = pltpu.roll(sx2_ref[pl.ds(r0, 1), :], lsh, axis=1)
        n_y2 = pltpu.roll(sy2_ref[pl.ds(r0, 1), :], lsh, axis=1)
        sn = jnp.max(n_sc[0:1, 0:1])
        return (pn, sn, ksn, kv, n_sc, n_x1, n_y1, n_x2, n_y2,
                kx1, ky1, kx2, ky2, ka, ocx, ocy, ow, oh, osc, ov)

    init = (jnp.int32(0), s0, jnp.int32(0), zi,
            key[0:1], x1[0:1], y1[0:1], x2[0:1], y2[0:1],
            zf, zf, zf, zf, zf, zf, zf, zf, zf, zf, zf)
    res = jax.lax.while_loop(cond, body, init)
    ocx_ref[...] = res[14]
    ocy_ref[...] = res[15]
    ow_ref[...] = res[16]
    oh_ref[...] = res[17]
    osc_ref[...] = res[18]
    ov_ref[...] = res[19]


def kernel(boxes, scores):
    pad = _NPAD - _N
    shp = (_NR, _NC)
    x1 = jnp.pad(boxes[:, 0], (0, pad)).reshape(shp)
    y1 = jnp.pad(boxes[:, 1], (0, pad)).reshape(shp)
    x2 = jnp.pad(boxes[:, 2], (0, pad)).reshape(shp)
    y2 = jnp.pad(boxes[:, 3], (0, pad)).reshape(shp)
    sc = jnp.pad(scores, (0, pad)).reshape(shp)
    outs = pl.pallas_call(
        _nms_body,
        out_shape=[jax.ShapeDtypeStruct((_BR, _BC), jnp.float32)] * 6,
        scratch_shapes=[pltpu.VMEM(shp, jnp.float32)] * 5,
    )(x1, y1, x2, y2, sc)
    cols = [o.reshape(-1)[:_MAXDET] for o in outs]
    return jnp.stack(cols, axis=-1)


# speedup vs baseline: 2.5167x; 1.4233x over previous
import jax
import jax.numpy as jnp
from jax.experimental import pallas as pl
from jax.experimental.pallas import tpu as pltpu

_N = 20000
_CONF = 0.35
_IOU = 0.5
_MAXDET = 1000
_NR, _NC = 256, 128
_BR, _BC = 8, 128
_BSZ = _BR * _BC
_NPAD = _NR * _NC


def _nms_body(x1_ref, y1_ref, x2_ref, y2_ref, sc_ref,
              ocx_ref, ocy_ref, ow_ref, oh_ref, osc_ref, ov_ref,
              k_ref, sx1_ref, sy1_ref, sx2_ref, sy2_ref):
    f = (jax.lax.broadcasted_iota(jnp.int32, (_NR, _NC), 0) * _NC
         + jax.lax.broadcasted_iota(jnp.int32, (_NR, _NC), 1))

    sc = sc_ref[...]
    key = jnp.where(sc > _CONF, sc, -1.0)
    idx = f
    x1 = x1_ref[...]
    y1 = y1_ref[...]
    x2 = x2_ref[...]
    y2 = y2_ref[...]

    def _exchange(s, kk, j, fetch):
        key, idx, x1, y1, x2, y2 = s
        lob = (f & j) == 0
        pk = fetch(key, lob)
        pi = fetch(idx, lob)
        pless = (pk > key) | ((pk == key) & (pi < idx))
        dirdesc = (f & kk) != 0
        take = jnp.logical_xor(jnp.logical_xor(pless, lob),
                               jnp.logical_not(dirdesc))
        return (jnp.where(take, pk, key),
                jnp.where(take, pi, idx),
                jnp.where(take, fetch(x1, lob), x1),
                jnp.where(take, fetch(y1, lob), y1),
                jnp.where(take, fetch(x2, lob), x2),
                jnp.where(take, fetch(y2, lob), y2))

    def _mk_branch(dr):
        def br(kk, *s):
            def fetch(x, lob):
                r = x.reshape(_NR // (2 * dr), 2, dr, _NC)
                return jnp.concatenate([r[:, 1:2], r[:, 0:1]],
                                       axis=1).reshape(_NR, _NC)

            return _exchange(s, kk, dr * _NC, fetch)

        return br

    _branches = [_mk_branch(1 << t) for t in range(8)]

    def _sub_body(c):
        kk, j, di = c[0], c[1], c[2]
        s = jax.lax.switch(di, _branches, kk, *c[3:])
        return (kk, jax.lax.shift_right_logical(j, 1), di - 1, *s)

    def _lane_body(c):
        kk, j = c[0], c[1]

        def fetch(x, lob):
            return jnp.where(lob, pltpu.roll(x, _NC - j, axis=1),
                             pltpu.roll(x, j, axis=1))

        return (kk, jax.lax.shift_right_logical(j, 1), c[2],
                *_exchange(c[3:], kk, j, fetch))

    def _level_body(lv, s):
        kk = jax.lax.shift_left(jnp.int32(1), lv)
        j0 = jax.lax.shift_right_logical(kk, 1)
        c = jax.lax.while_loop(lambda t: t[1] >= _NC, _sub_body,
                               (kk, j0, lv - 8) + s)
        c = jax.lax.while_loop(lambda t: t[1] >= 1, _lane_body, c)
        return c[3:]

    res = jax.lax.fori_loop(1, 16, _level_body,
                            (key, idx, x1, y1, x2, y2))
    key, _, x1, y1, x2, y2 = res

    k_ref[...] = key
    sx1_ref[...] = x1
    sy1_ref[...] = y1
    sx2_ref[...] = x2
    sy2_ref[...] = y2

    _U = 8
    g = (jax.lax.broadcasted_iota(jnp.int32, (_BR, _BC), 0) * _BC
         + jax.lax.broadcasted_iota(jnp.int32, (_BR, _BC), 1))
    zf = jnp.zeros((_BR, _BC), jnp.float32)
    zi = jnp.zeros((_BR, _BC), jnp.int32)
    ones_l = jnp.ones((_BC, _BC), jnp.float32)
    ones_s = jnp.ones((_BR, _BR), jnp.float32)

    sel0 = f == 0
    s0 = jnp.sum(jnp.where(sel0, key, 0.0))

    def cond(c):
        return (c[1] > 0.0) & (c[2] < _MAXDET)

    def body(c):
        (p, s_next, ks, kv, c_sc, c_x1, c_y1, c_x2, c_y2,
         kx1, ky1, kx2, ky2, ka, ocx, ocy, ow, oh, osc, ov) = c

        def ext(w, u):
            return jnp.broadcast_to(w[0:1, u:u + 1], (_BR, _BC))

        ca = (c_x2 - c_x1) * (c_y2 - c_y1)
        pre = []
        for u in range(_U):
            sv = ext(c_sc, u)
            bx1 = ext(c_x1, u)
            by1 = ext(c_y1, u)
            bx2 = ext(c_x2, u)
            by2 = ext(c_y2, u)
            a1 = (bx2 - bx1) * (by2 - by1)
            ix1 = jnp.maximum(bx1, kx1)
            iy1 = jnp.maximum(by1, ky1)
            ix2 = jnp.minimum(bx2, kx2)
            iy2 = jnp.minimum(by2, ky2)
            inter = (jnp.maximum(ix2 - ix1, 0.0)
                     * jnp.maximum(iy2 - iy1, 0.0))
            den = a1 + ka - inter + 1e-9
            gt = jnp.where(inter - den * 0.5 > den * (2.0 ** -25),
                           1.0, 0.0)
            rs = jax.lax.dot_general(
                gt, ones_l, (((1,), (0,)), ((), ())),
                preferred_element_type=jnp.float32)
            t = jax.lax.dot_general(
                ones_s, rs, (((1,), (0,)), ((), ())),
                preferred_element_type=jnp.float32)
            px1 = jnp.maximum(bx1[0:1, :], c_x1)
            py1 = jnp.maximum(by1[0:1, :], c_y1)
            px2 = jnp.minimum(bx2[0:1, :], c_x2)
            py2 = jnp.minimum(by2[0:1, :], c_y2)
            pin = (jnp.maximum(px2 - px1, 0.0)
                   * jnp.maximum(py2 - py1, 0.0))
            pden = a1[0:1, :] + ca - pin + 1e-9
            pgt = pin - pden * 0.5 > pden * (2.0 ** -25)
            pre.append((sv, bx1, by1, bx2, by2, a1, t, pgt))

        supp = jnp.zeros((1, _NC), jnp.bool_)
        for u in range(_U):
            sv, bx1, by1, bx2, by2, a1, t, pgt = pre[u]
            sab = jnp.broadcast_to(supp[0:1, u:u + 1], (_BR, _BC))
            okv = ((t < 0.5) & jnp.logical_not(sab)
                   & (sv > 0.0) & (kv < _MAXDET))
            slot = (g == kv) & okv
            kx1 = jnp.where(slot, bx1, kx1)
            ky1 = jnp.where(slot, by1, ky1)
            kx2 = jnp.where(slot, bx2, kx2)
            ky2 = jnp.where(slot, by2, ky2)
            ka = jnp.where(slot, a1, ka)
            w = bx2 - bx1
            h = by2 - by1
            ocx = jnp.where(slot, bx1 + w / 2.0, ocx)
            ocy = jnp.where(slot, by1 + h / 2.0, ocy)
            ow = jnp.where(slot, w, ow)
            oh = jnp.where(slot, h, oh)
            osc = jnp.where(slot, sv, osc)
            ov = jnp.where(slot, 1.0, ov)
            kv = kv + jnp.where(okv, 1, 0)
            supp = supp | (okv[0:1, :] & pgt)

        pn = p + _U
        ksn = jnp.max(kv[0:1, 0:1])
        r0 = jax.lax.shift_right_logical(pn, 7)
        lsh = (_NC - (pn & (_NC - 1))) & (_NC - 1)
        n_sc = pltpu.roll(k_ref[pl.ds(r0, 1), :], lsh, axis=1)
        n_x1 = pltpu.roll(sx1_ref[pl.ds(r0, 1), :], lsh, axis=1)
        n_y1 = pltpu.roll(sy1_ref[pl.ds(r0, 1), :], lsh, axis=1)
        n_x2 = pltpu.roll(sx2_ref[pl.ds(r0, 1), :], lsh, axis=1)
        n_y2 = pltpu.roll(sy2_ref[pl.ds(r0, 1), :], lsh, axis=1)
        sn = jnp.max(n_sc[0:1, 0:1])
        return (pn, sn, ksn, kv, n_sc, n_x1, n_y1, n_x2, n_y2,
                kx1, ky1, kx2, ky2, ka, ocx, ocy, ow, oh, osc, ov)

    init = (jnp.int32(0), s0, jnp.int32(0), zi,
            key[0:1], x1[0:1], y1[0:1], x2[0:1], y2[0:1],
            zf, zf, zf, zf, zf, zf, zf, zf, zf, zf, zf)
    res = jax.lax.while_loop(cond, body, init)
    ocx_ref[...] = res[14]
    ocy_ref[...] = res[15]
    ow_ref[...] = res[16]
    oh_ref[...] = res[17]
    osc_ref[...] = res[18]
    ov_ref[...] = res[19]


def kernel(boxes, scores):
    pad = _NPAD - _N
    shp = (_NR, _NC)
    x1 = jnp.pad(boxes[:, 0], (0, pad)).reshape(shp)
    y1 = jnp.pad(boxes[:, 1], (0, pad)).reshape(shp)
    x2 = jnp.pad(boxes[:, 2], (0, pad)).reshape(shp)
    y2 = jnp.pad(boxes[:, 3], (0, pad)).reshape(shp)
    sc = jnp.pad(scores, (0, pad)).reshape(shp)
    outs = pl.pallas_call(
        _nms_body,
        out_shape=[jax.ShapeDtypeStruct((_BR, _BC), jnp.float32)] * 6,
        scratch_shapes=[pltpu.VMEM(shp, jnp.float32)] * 5,
    )(x1, y1, x2, y2, sc)
    cols = [o.reshape(-1)[:_MAXDET] for o in outs]
    return jnp.stack(cols, axis=-1)


# speedup vs baseline: 2.5667x; 1.0199x over previous
import jax
import jax.numpy as jnp
from jax.experimental import pallas as pl
from jax.experimental.pallas import tpu as pltpu

_N = 20000
_CONF = 0.35
_IOU = 0.5
_MAXDET = 1000
_NR, _NC = 256, 128
_BR, _BC = 8, 128
_BSZ = _BR * _BC
_NPAD = _NR * _NC


def _nms_body(x1_ref, y1_ref, x2_ref, y2_ref, sc_ref,
              ocx_ref, ocy_ref, ow_ref, oh_ref, osc_ref, ov_ref,
              k_ref, sx1_ref, sy1_ref, sx2_ref, sy2_ref):
    f = (jax.lax.broadcasted_iota(jnp.int32, (_NR, _NC), 0) * _NC
         + jax.lax.broadcasted_iota(jnp.int32, (_NR, _NC), 1))

    sc = sc_ref[...]
    key = jnp.where(sc > _CONF, sc, -1.0)
    idx = f
    x1 = x1_ref[...]
    y1 = y1_ref[...]
    x2 = x2_ref[...]
    y2 = y2_ref[...]

    def _exchange(s, kk, j, fetch):
        key, idx, x1, y1, x2, y2 = s
        lob = (f & j) == 0
        pk = fetch(key, lob)
        pi = fetch(idx, lob)
        pless = (pk > key) | ((pk == key) & (pi < idx))
        dirdesc = (f & kk) != 0
        take = jnp.logical_xor(jnp.logical_xor(pless, lob),
                               jnp.logical_not(dirdesc))
        return (jnp.where(take, pk, key),
                jnp.where(take, pi, idx),
                jnp.where(take, fetch(x1, lob), x1),
                jnp.where(take, fetch(y1, lob), y1),
                jnp.where(take, fetch(x2, lob), x2),
                jnp.where(take, fetch(y2, lob), y2))

    def _mk_branch(dr):
        def br(kk, *s):
            def fetch(x, lob):
                r = x.reshape(_NR // (2 * dr), 2, dr, _NC)
                return jnp.concatenate([r[:, 1:2], r[:, 0:1]],
                                       axis=1).reshape(_NR, _NC)

            return _exchange(s, kk, dr * _NC, fetch)

        return br

    _branches = [_mk_branch(1 << t) for t in range(8)]

    def _sub_body(c):
        kk, j, di = c[0], c[1], c[2]
        s = jax.lax.switch(di, _branches, kk, *c[3:])
        return (kk, jax.lax.shift_right_logical(j, 1), di - 1, *s)

    def _lane_body(c):
        kk, j = c[0], c[1]

        def fetch(x, lob):
            return jnp.where(lob, pltpu.roll(x, _NC - j, axis=1),
                             pltpu.roll(x, j, axis=1))

        return (kk, jax.lax.shift_right_logical(j, 1), c[2],
                *_exchange(c[3:], kk, j, fetch))

    def _level_body(lv, s):
        kk = jax.lax.shift_left(jnp.int32(1), lv)
        j0 = jax.lax.shift_right_logical(kk, 1)
        c = jax.lax.while_loop(lambda t: t[1] >= _NC, _sub_body,
                               (kk, j0, lv - 8) + s)
        c = jax.lax.while_loop(lambda t: t[1] >= 1, _lane_body, c)
        return c[3:]

    res = jax.lax.fori_loop(1, 16, _level_body,
                            (key, idx, x1, y1, x2, y2))
    key, _, x1, y1, x2, y2 = res

    k_ref[...] = key
    sx1_ref[...] = x1
    sy1_ref[...] = y1
    sx2_ref[...] = x2
    sy2_ref[...] = y2

    _U = 16
    g = (jax.lax.broadcasted_iota(jnp.int32, (_BR, _BC), 0) * _BC
         + jax.lax.broadcasted_iota(jnp.int32, (_BR, _BC), 1))
    zf = jnp.zeros((_BR, _BC), jnp.float32)
    zi = jnp.zeros((_BR, _BC), jnp.int32)
    ones_l = jnp.ones((_BC, _BC), jnp.float32)
    ones_s = jnp.ones((_BR, _BR), jnp.float32)

    sel0 = f == 0
    s0 = jnp.sum(jnp.where(sel0, key, 0.0))

    def cond(c):
        return (c[1] > 0.0) & (c[2] < _MAXDET)

    def body(c):
        (p, s_next, ks, kv, c_sc, c_x1, c_y1, c_x2, c_y2,
         kx1, ky1, kx2, ky2, ka, ocx, ocy, ow, oh, osc, ov) = c

        def ext(w, u):
            return jnp.broadcast_to(w[0:1, u:u + 1], (_BR, _BC))

        ca = (c_x2 - c_x1) * (c_y2 - c_y1)
        pre = []
        for u in range(_U):
            sv = ext(c_sc, u)
            bx1 = ext(c_x1, u)
            by1 = ext(c_y1, u)
            bx2 = ext(c_x2, u)
            by2 = ext(c_y2, u)
            a1 = (bx2 - bx1) * (by2 - by1)
            ix1 = jnp.maximum(bx1, kx1)
            iy1 = jnp.maximum(by1, ky1)
            ix2 = jnp.minimum(bx2, kx2)
            iy2 = jnp.minimum(by2, ky2)
            inter = (jnp.maximum(ix2 - ix1, 0.0)
                     * jnp.maximum(iy2 - iy1, 0.0))
            den = a1 + ka - inter + 1e-9
            gt = jnp.where(inter - den * 0.5 > den * (2.0 ** -25),
                           1.0, 0.0)
            rs = jax.lax.dot_general(
                gt, ones_l, (((1,), (0,)), ((), ())),
                preferred_element_type=jnp.float32)
            t = jax.lax.dot_general(
                ones_s, rs, (((1,), (0,)), ((), ())),
                preferred_element_type=jnp.float32)
            px1 = jnp.maximum(bx1[0:1, :], c_x1)
            py1 = jnp.maximum(by1[0:1, :], c_y1)
            px2 = jnp.minimum(bx2[0:1, :], c_x2)
            py2 = jnp.minimum(by2[0:1, :], c_y2)
            pin = (jnp.maximum(px2 - px1, 0.0)
                   * jnp.maximum(py2 - py1, 0.0))
            pden = a1[0:1, :] + ca - pin + 1e-9
            pgt = pin - pden * 0.5 > pden * (2.0 ** -25)
            pre.append((sv, bx1, by1, bx2, by2, a1, t, pgt))

        supp = jnp.zeros((1, _NC), jnp.bool_)
        for u in range(_U):
            sv, bx1, by1, bx2, by2, a1, t, pgt = pre[u]
            sab = jnp.broadcast_to(supp[0:1, u:u + 1], (_BR, _BC))
            okv = ((t < 0.5) & jnp.logical_not(sab)
                   & (sv > 0.0) & (kv < _MAXDET))
            slot = (g == kv) & okv
            kx1 = jnp.where(slot, bx1, kx1)
            ky1 = jnp.where(slot, by1, ky1)
            kx2 = jnp.where(slot, bx2, kx2)
            ky2 = jnp.where(slot, by2, ky2)
            ka = jnp.where(slot, a1, ka)
            w = bx2 - bx1
            h = by2 - by1
            ocx = jnp.where(slot, bx1 + w / 2.0, ocx)
            ocy = jnp.where(slot, by1 + h / 2.0, ocy)
            ow = jnp.where(slot, w, ow)
            oh = jnp.where(slot, h, oh)
            osc = jnp.where(slot, sv, osc)
            ov = jnp.where(slot, 1.0, ov)
            kv = kv + jnp.where(okv, 1, 0)
            supp = supp | (okv[0:1, :] & pgt)

        pn = p + _U
        ksn = jnp.max(kv[0:1, 0:1])
        r0 = jax.lax.shift_right_logical(pn, 7)
        lsh = (_NC - (pn & (_NC - 1))) & (_NC - 1)
        n_sc = pltpu.roll(k_ref[pl.ds(r0, 1), :], lsh, axis=1)
        n_x1 = pltpu.roll(sx1_ref[pl.ds(r0, 1), :], lsh, axis=1)
        n_y1 = pltpu.roll(sy1_ref[pl.ds(r0, 1), :], lsh, axis=1)
        n_x2 = pltpu.roll(sx2_ref[pl.ds(r0, 1), :], lsh, axis=1)
        n_y2 = pltpu.roll(sy2_ref[pl.ds(r0, 1), :], lsh, axis=1)
        sn = jnp.max(n_sc[0:1, 0:1])
        return (pn, sn, ksn, kv, n_sc, n_x1, n_y1, n_x2, n_y2,
                kx1, ky1, kx2, ky2, ka, ocx, ocy, ow, oh, osc, ov)

    init = (jnp.int32(0), s0, jnp.int32(0), zi,
            key[0:1], x1[0:1], y1[0:1], x2[0:1], y2[0:1],
            zf, zf, zf, zf, zf, zf, zf, zf, zf, zf, zf)
    res = jax.lax.while_loop(cond, body, init)
    ocx_ref[...] = res[14]
    ocy_ref[...] = res[15]
    ow_ref[...] = res[16]
    oh_ref[...] = res[17]
    osc_ref[...] = res[18]
    ov_ref[...] = res[19]


def kernel(boxes, scores):
    pad = _NPAD - _N
    shp = (_NR, _NC)
    x1 = jnp.pad(boxes[:, 0], (0, pad)).reshape(shp)
    y1 = jnp.pad(boxes[:, 1], (0, pad)).reshape(shp)
    x2 = jnp.pad(boxes[:, 2], (0, pad)).reshape(shp)
    y2 = jnp.pad(boxes[:, 3], (0, pad)).reshape(shp)
    sc = jnp.pad(scores, (0, pad)).reshape(shp)
    outs = pl.pallas_call(
        _nms_body,
        out_shape=[jax.ShapeDtypeStruct((_BR, _BC), jnp.float32)] * 6,
        scratch_shapes=[pltpu.VMEM(shp, jnp.float32)] * 5,
    )(x1, y1, x2, y2, sc)
    cols = [o.reshape(-1)[:_MAXDET] for o in outs]
    return jnp.stack(cols, axis=-1)


# speedup vs baseline: 2.6881x; 1.0473x over previous
import jax
import jax.numpy as jnp
from jax.experimental import pallas as pl
from jax.experimental.pallas import tpu as pltpu

_N = 20000
_CONF = 0.35
_IOU = 0.5
_MAXDET = 1000
_NR, _NC = 256, 128
_BR, _BC = 8, 128
_BSZ = _BR * _BC
_NPAD = _NR * _NC


def _nms_body(x1_ref, y1_ref, x2_ref, y2_ref, sc_ref,
              ocx_ref, ocy_ref, ow_ref, oh_ref, osc_ref, ov_ref,
              k_ref, sx1_ref, sy1_ref, sx2_ref, sy2_ref):
    f = (jax.lax.broadcasted_iota(jnp.int32, (_NR, _NC), 0) * _NC
         + jax.lax.broadcasted_iota(jnp.int32, (_NR, _NC), 1))

    sc = sc_ref[...]
    key = jnp.where(sc > _CONF, sc, -1.0)
    idx = f
    x1 = x1_ref[...]
    y1 = y1_ref[...]
    x2 = x2_ref[...]
    y2 = y2_ref[...]

    def _exchange(s, kk, j, fetch):
        key, idx, x1, y1, x2, y2 = s
        lob = (f & j) == 0
        pk = fetch(key, lob)
        pi = fetch(idx, lob)
        pless = (pk > key) | ((pk == key) & (pi < idx))
        dirdesc = (f & kk) != 0
        take = jnp.logical_xor(jnp.logical_xor(pless, lob),
                               jnp.logical_not(dirdesc))
        return (jnp.where(take, pk, key),
                jnp.where(take, pi, idx),
                jnp.where(take, fetch(x1, lob), x1),
                jnp.where(take, fetch(y1, lob), y1),
                jnp.where(take, fetch(x2, lob), x2),
                jnp.where(take, fetch(y2, lob), y2))

    def _mk_branch(dr):
        def br(kk, *s):
            def fetch(x, lob):
                r = x.reshape(_NR // (2 * dr), 2, dr, _NC)
                return jnp.concatenate([r[:, 1:2], r[:, 0:1]],
                                       axis=1).reshape(_NR, _NC)

            return _exchange(s, kk, dr * _NC, fetch)

        return br

    _branches = [_mk_branch(1 << t) for t in range(8)]

    def _sub_body(c):
        kk, j, di = c[0], c[1], c[2]
        s = jax.lax.switch(di, _branches, kk, *c[3:])
        return (kk, jax.lax.shift_right_logical(j, 1), di - 1, *s)

    def _lane_body(c):
        kk, j = c[0], c[1]

        def fetch(x, lob):
            return jnp.where(lob, pltpu.roll(x, _NC - j, axis=1),
                             pltpu.roll(x, j, axis=1))

        return (kk, jax.lax.shift_right_logical(j, 1), c[2],
                *_exchange(c[3:], kk, j, fetch))

    def _level_body(lv, s):
        kk = jax.lax.shift_left(jnp.int32(1), lv)
        j0 = jax.lax.shift_right_logical(kk, 1)
        c = jax.lax.while_loop(lambda t: t[1] >= _NC, _sub_body,
                               (kk, j0, lv - 8) + s)
        c = jax.lax.while_loop(lambda t: t[1] >= 1, _lane_body, c)
        return c[3:]

    res = jax.lax.fori_loop(1, 16, _level_body,
                            (key, idx, x1, y1, x2, y2))
    key, _, x1, y1, x2, y2 = res

    k_ref[...] = key
    sx1_ref[...] = x1
    sy1_ref[...] = y1
    sx2_ref[...] = x2
    sy2_ref[...] = y2

    _U = 16
    g = (jax.lax.broadcasted_iota(jnp.int32, (_BR, _BC), 0) * _BC
         + jax.lax.broadcasted_iota(jnp.int32, (_BR, _BC), 1))
    zf = jnp.zeros((_BR, _BC), jnp.float32)
    zi = jnp.zeros((_BR, _BC), jnp.int32)
    ones_l = jnp.ones((_BC, _BC), jnp.float32)
    bd0 = jax.lax.broadcasted_iota(jnp.int32, (_BC, _BC), 0)
    bd1 = jax.lax.broadcasted_iota(jnp.int32, (_BC, _BC), 1)
    bdiag = jnp.where(jax.lax.shift_right_logical(bd0, 3)
                      == jax.lax.shift_right_logical(bd1, 3), 1.0, 0.0)

    sel0 = f == 0
    s0 = jnp.sum(jnp.where(sel0, key, 0.0))

    def cond(c):
        return (c[1] > 0.0) & (c[2] < _MAXDET)

    def body(c):
        (p, s_next, ks, kv, c_sc, c_x1, c_y1, c_x2, c_y2,
         kx1, ky1, kx2, ky2, ka, ocx, ocy, ow, oh, osc, ov) = c

        def ext(w, u):
            return jnp.broadcast_to(w[0:1, u:u + 1], (_BR, _BC))

        ca = (c_x2 - c_x1) * (c_y2 - c_y1)
        pre = []
        gts = []
        for u in range(_U):
            sv = ext(c_sc, u)
            bx1 = ext(c_x1, u)
            by1 = ext(c_y1, u)
            bx2 = ext(c_x2, u)
            by2 = ext(c_y2, u)
            a1 = (bx2 - bx1) * (by2 - by1)
            ix1 = jnp.maximum(bx1, kx1)
            iy1 = jnp.maximum(by1, ky1)
            ix2 = jnp.minimum(bx2, kx2)
            iy2 = jnp.minimum(by2, ky2)
            inter = (jnp.maximum(ix2 - ix1, 0.0)
                     * jnp.maximum(iy2 - iy1, 0.0))
            den = a1 + ka - inter + 1e-9
            gt = jnp.where(inter - den * 0.5 > den * (2.0 ** -25),
                           1.0, 0.0)
            gts.append(gt)
            px1 = jnp.maximum(bx1[0:1, :], c_x1)
            py1 = jnp.maximum(by1[0:1, :], c_y1)
            px2 = jnp.minimum(bx2[0:1, :], c_x2)
            py2 = jnp.minimum(by2[0:1, :], c_y2)
            pin = (jnp.maximum(px2 - px1, 0.0)
                   * jnp.maximum(py2 - py1, 0.0))
            pden = a1[0:1, :] + ca - pin + 1e-9
            pgt = pin - pden * 0.5 > pden * (2.0 ** -25)
            pre.append((sv, bx1, by1, bx2, by2, a1, pgt))

        pack = jnp.concatenate(gts, axis=0)
        ps = jax.lax.dot_general(
            pack, ones_l, (((1,), (0,)), ((), ())),
            preferred_element_type=jnp.float32)
        tt = jax.lax.dot_general(
            bdiag, ps, (((1,), (0,)), ((), ())),
            preferred_element_type=jnp.float32)

        supp = jnp.zeros((1, _NC), jnp.bool_)
        for u in range(_U):
            sv, bx1, by1, bx2, by2, a1, pgt = pre[u]
            t = tt[_BR * u:_BR * u + _BR, :]
            sab = jnp.broadcast_to(supp[0:1, u:u + 1], (_BR, _BC))
            okv = ((t < 0.5) & jnp.logical_not(sab)
                   & (sv > 0.0) & (kv < _MAXDET))
            slot = (g == kv) & okv
            kx1 = jnp.where(slot, bx1, kx1)
            ky1 = jnp.where(slot, by1, ky1)
            kx2 = jnp.where(slot, bx2, kx2)
            ky2 = jnp.where(slot, by2, ky2)
            ka = jnp.where(slot, a1, ka)
            w = bx2 - bx1
            h = by2 - by1
            ocx = jnp.where(slot, bx1 + w / 2.0, ocx)
            ocy = jnp.where(slot, by1 + h / 2.0, ocy)
            ow = jnp.where(slot, w, ow)
            oh = jnp.where(slot, h, oh)
            osc = jnp.where(slot, sv, osc)
            ov = jnp.where(slot, 1.0, ov)
            kv = kv + jnp.where(okv, 1, 0)
            supp = supp | (okv[0:1, :] & pgt)

        pn = p + _U
        ksn = jnp.max(kv[0:1, 0:1])
        r0 = jax.lax.shift_right_logical(pn, 7)
        lsh = (_NC - (pn & (_NC - 1))) & (_NC - 1)
        n_sc = pltpu.roll(k_ref[pl.ds(r0, 1), :], lsh, axis=1)
        n_x1 = pltpu.roll(sx1_ref[pl.ds(r0, 1), :], lsh, axis=1)
        n_y1 = pltpu.roll(sy1_ref[pl.ds(r0, 1), :], lsh, axis=1)
        n_x2 = pltpu.roll(sx2_ref[pl.ds(r0, 1), :], lsh, axis=1)
        n_y2 = pltpu.roll(sy2_ref[pl.ds(r0, 1), :], lsh, axis=1)
        sn = jnp.max(n_sc[0:1, 0:1])
        return (pn, sn, ksn, kv, n_sc, n_x1, n_y1, n_x2, n_y2,
                kx1, ky1, kx2, ky2, ka, ocx, ocy, ow, oh, osc, ov)

    init = (jnp.int32(0), s0, jnp.int32(0), zi,
            key[0:1], x1[0:1], y1[0:1], x2[0:1], y2[0:1],
            zf, zf, zf, zf, zf, zf, zf, zf, zf, zf, zf)
    res = jax.lax.while_loop(cond, body, init)
    ocx_ref[...] = res[14]
    ocy_ref[...] = res[15]
    ow_ref[...] = res[16]
    oh_ref[...] = res[17]
    osc_ref[...] = res[18]
    ov_ref[...] = res[19]


def kernel(boxes, scores):
    pad = _NPAD - _N
    shp = (_NR, _NC)
    x1 = jnp.pad(boxes[:, 0], (0, pad)).reshape(shp)
    y1 = jnp.pad(boxes[:, 1], (0, pad)).reshape(shp)
    x2 = jnp.pad(boxes[:, 2], (0, pad)).reshape(shp)
    y2 = jnp.pad(boxes[:, 3], (0, pad)).reshape(shp)
    sc = jnp.pad(scores, (0, pad)).reshape(shp)
    outs = pl.pallas_call(
        _nms_body,
        out_shape=[jax.ShapeDtypeStruct((_BR, _BC), jnp.float32)] * 6,
        scratch_shapes=[pltpu.VMEM(shp, jnp.float32)] * 5,
    )(x1, y1, x2, y2, sc)
    cols = [o.reshape(-1)[:_MAXDET] for o in outs]
    return jnp.stack(cols, axis=-1)
